# Initial kernel scaffold; baseline (speedup 1.0000x reference)
import jax
import jax.numpy as jnp
from jax.experimental import pallas as pl


def kernel(queries, keys, values):
    raise NotImplementedError("write your pallas kernel here")


# speedup vs baseline: 5.3541x; 5.3541x over previous
import functools
from math import sqrt

import jax
import jax.numpy as jnp
from jax import lax
from jax.experimental import pallas as pl
from jax.experimental.pallas import tpu as pltpu
from jax.experimental.pallas import tpu_sc as plsc

_CLUSTERS = 128
_ITERATIONS = 10
_BITS = 32
_BP = 40
_HIGH = lax.Precision.HIGHEST


def _tc_body(q_ref, k_ref, v_ref, planes_ref, bias_ref, sel_ref, vc_ref, assign_ref):
    L, E = q_ref.shape[1], q_ref.shape[2]
    C = _CLUSTERS
    q = q_ref[0]
    k = k_ref[0]
    v = v_ref[0]

    proj = jnp.dot(q.astype(jnp.bfloat16), planes_ref[...],
                   preferred_element_type=jnp.float32) + bias_ref[0:1, :]
    bits = (proj > 0.0).astype(jnp.float32)
    bits_bf = bits.astype(jnp.bfloat16)
    bpm_bf = (bits * 2.0 - 1.0).astype(jnp.bfloat16)

    cb = jnp.dot(sel_ref[...], bits_bf,
                 preferred_element_type=jnp.float32)

    lane = lax.broadcasted_iota(jnp.int32, (C, _BP), 1)
    rowc = lax.broadcasted_iota(jnp.int32, (C, _BP), 0).astype(jnp.float32)
    aux = jnp.where(lane == _BITS, -rowc, 0.0)
    is_bit = lane < _BITS

    def _key_onehot(cb):
        cpm_aug = jnp.where(is_bit, cb * 256.0 - 128.0, aux)
        key = lax.dot_general(cpm_aug.astype(jnp.bfloat16), bpm_bf,
                              (((1,), (1,)), ((), ())),
                              preferred_element_type=jnp.float32)
        m = jnp.max(key, axis=0, keepdims=True)
        return m, (key == m)

    def _iter(_, cb):
        _, hit = _key_onehot(cb)
        bs = jnp.dot(hit.astype(jnp.bfloat16), bits_bf,
                     preferred_element_type=jnp.float32)
        counts = bs[:, _BITS:_BITS + 1]
        maj = (bs * 2.0 > counts).astype(jnp.float32)
        return jnp.where(counts > 0.0, maj, cb)

    cb = lax.fori_loop(0, _ITERATIONS, _iter, cb)

    m, hit = _key_onehot(cb)
    onehot_bf = hit.astype(jnp.bfloat16)
    bs = jnp.dot(onehot_bf, bits_bf, preferred_element_type=jnp.float32)
    counts = jnp.maximum(bs[:, _BITS:_BITS + 1], 1.0)
    assign = (-m.astype(jnp.int32)) % C

    q_sum = lax.dot_general(hit.astype(jnp.float32), q, (((1,), (0,)), ((), ())),
                            precision=_HIGH, preferred_element_type=jnp.float32)
    qg = (q_sum / counts).astype(jnp.bfloat16)
    temp = jnp.float32(1.0 / sqrt(E))
    logits = lax.dot_general(qg, k.astype(jnp.bfloat16), (((1,), (1,)), ((), ())),
                             preferred_element_type=jnp.float32) * temp
    lmax = jnp.max(logits, axis=1, keepdims=True)
    p = jnp.exp(logits - lmax)
    a = p / jnp.sum(p, axis=1, keepdims=True)
    vc = jnp.dot(a.astype(jnp.bfloat16), v.astype(jnp.bfloat16),
                 preferred_element_type=jnp.float32)

    vc_ref[0] = vc
    assign_ref[0] = assign


def _tc_cluster_attend(q, k, v, planes_aug, bias_aug, sel):
    NH, L, E = q.shape
    return pl.pallas_call(
        _tc_body,
        grid=(NH,),
        in_specs=[
            pl.BlockSpec((1, L, E), lambda i: (i, 0, 0)),
            pl.BlockSpec((1, L, E), lambda i: (i, 0, 0)),
            pl.BlockSpec((1, L, E), lambda i: (i, 0, 0)),
            pl.BlockSpec((E, _BP), lambda i: (0, 0)),
            pl.BlockSpec((8, _BP), lambda i: (0, 0)),
            pl.BlockSpec((_CLUSTERS, L), lambda i: (0, 0)),
        ],
        out_specs=[
            pl.BlockSpec((1, _CLUSTERS, E), lambda i: (i, 0, 0)),
            pl.BlockSpec((1, 1, L), lambda i: (i, 0, 0)),
        ],
        out_shape=[
            jax.ShapeDtypeStruct((NH, _CLUSTERS, E), jnp.float32),
            jax.ShapeDtypeStruct((NH, 1, L), jnp.int32),
        ],
    )(q, k, v, planes_aug, bias_aug, sel)


def _make_sc_gather(B, D):
    info = plsc.get_sparse_core_info()
    NC, NS = info.num_cores, info.num_subcores
    NW = NC * NS
    RPW = B // NW
    CH = 128
    NBUF = 4
    nch = RPW // CH
    mesh = plsc.VectorSubcoreMesh(core_axis_name="c", subcore_axis_name="s")

    @functools.partial(
        pl.kernel,
        mesh=mesh,
        compiler_params=pltpu.CompilerParams(use_tc_tiling_on_sc=False),
        out_type=jax.ShapeDtypeStruct((B, D), jnp.float32),
        scratch_types=[
            pltpu.VMEM((RPW,), jnp.int32),
            pltpu.VMEM((NBUF, CH, D), jnp.float32),
        ] + [pltpu.SemaphoreType.DMA] * (2 * NBUF),
    )
    def _sc_gather(table_hbm, idx_hbm, out_hbm, idx_v, rows_v, *sems):
        gsems, wsems = sems[:NBUF], sems[NBUF:]
        wid = lax.axis_index("s") * NC + lax.axis_index("c")
        base = wid * RPW
        pltpu.sync_copy(idx_hbm.at[pl.ds(base, RPW)], idx_v)

        def _start(c):
            return pltpu.async_copy(
                table_hbm.at[idx_v.at[pl.ds(c * CH, CH)]],
                rows_v.at[c % NBUF], gsems[c % NBUF])

        handles = {}
        wh = {}
        for b in range(min(NBUF, nch)):
            handles[b] = _start(b)
        for c in range(nch):
            handles.pop(c).wait()
            wh[c] = pltpu.async_copy(
                rows_v.at[c % NBUF],
                out_hbm.at[pl.ds(base + c * CH, CH)], wsems[c % NBUF])
            nxt = c + NBUF
            if nxt < nch:
                wh.pop(nxt - NBUF).wait()
                handles[nxt] = _start(nxt)
        for c in sorted(wh):
            wh.pop(c).wait()

    return _sc_gather


def kernel(queries, keys, values):
    N, L, H, E = queries.shape
    D = values.shape[-1]
    NH = N * H
    B = N * L * H

    q = jnp.transpose(queries, (0, 2, 1, 3)).reshape(NH, L, E)
    k = jnp.transpose(keys, (0, 2, 1, 3)).reshape(NH, L, E)
    v = jnp.transpose(values, (0, 2, 1, 3)).reshape(NH, L, D)

    planes = jax.random.normal(jax.random.key(42), (_BITS, E + 1), dtype=jnp.float32)
    pad = _BP - _BITS
    planes_aug = jnp.concatenate(
        [planes[:, :-1].T, jnp.zeros((E, pad), jnp.float32)], axis=1
    ).astype(jnp.bfloat16)
    bias_aug = jnp.tile(
        jnp.concatenate([planes[:, -1], jnp.ones((pad,), jnp.float32)])[None, :],
        (8, 1))
    init_idx = jnp.linspace(0, L - 1, _CLUSTERS).astype(jnp.int32)
    sel = (init_idx[:, None] == jnp.arange(L, dtype=jnp.int32)[None, :]
           ).astype(jnp.bfloat16)

    vc, assign = _tc_cluster_attend(q, k, v, planes_aug, bias_aug, sel)

    head_off = (jnp.arange(NH, dtype=jnp.int32) * _CLUSTERS).reshape(N, H, 1)
    idx = jnp.transpose(assign.reshape(N, H, L) + head_off, (0, 2, 1)).reshape(B)

    out = _make_sc_gather(B, D)(vc.reshape(NH * _CLUSTERS, D), idx)
    return out.reshape(N, L, H, D)


# speedup vs baseline: 5.5073x; 1.0286x over previous
import functools
from math import sqrt

import jax
import jax.numpy as jnp
from jax import lax
from jax.experimental import pallas as pl
from jax.experimental.pallas import tpu as pltpu
from jax.experimental.pallas import tpu_sc as plsc

_CLUSTERS = 128
_ITERATIONS = 10
_BITS = 32
_BP = 40
_HIGH = lax.Precision.HIGHEST


def _tc_body(q_ref, k_ref, v_ref, planes_ref, bias_ref, sel_ref, vc_ref, assign_ref):
    L, E = q_ref.shape[1], q_ref.shape[2]
    C = _CLUSTERS
    q = q_ref[0]
    k = k_ref[0]
    v = v_ref[0]

    proj = jnp.dot(q, planes_ref[...],
                   preferred_element_type=jnp.float32) + bias_ref[0:1, :]
    bits = (proj > 0.0).astype(jnp.float32)
    bpm = bits * 2.0 - 1.0

    cb = jnp.dot(sel_ref[...], bits,
                 preferred_element_type=jnp.float32)

    lane = lax.broadcasted_iota(jnp.int32, (C, _BP), 1)
    rowc = lax.broadcasted_iota(jnp.int32, (C, _BP), 0).astype(jnp.float32)
    aux = jnp.where(lane == _BITS, -rowc, 0.0)
    is_bit = lane < _BITS

    def _key_onehot(cb):
        cpm_aug = jnp.where(is_bit, cb * 256.0 - 128.0, aux)
        key = lax.dot_general(cpm_aug, bpm,
                              (((1,), (1,)), ((), ())),
                              preferred_element_type=jnp.float32)
        m = jnp.max(key, axis=0, keepdims=True)
        return m, (key == m)

    def _iter(_, cb):
        _, hit = _key_onehot(cb)
        bs = jnp.dot(hit.astype(jnp.float32), bits,
                     preferred_element_type=jnp.float32)
        counts = bs[:, _BITS:_BITS + 1]
        maj = (bs * 2.0 > counts).astype(jnp.float32)
        return jnp.where(counts > 0.0, maj, cb)

    cb = lax.fori_loop(0, _ITERATIONS, _iter, cb)

    m, hit = _key_onehot(cb)
    onehot = hit.astype(jnp.float32)
    bs = jnp.dot(onehot, bits, preferred_element_type=jnp.float32)
    counts = jnp.maximum(bs[:, _BITS:_BITS + 1], 1.0)
    assign = (-m.astype(jnp.int32)) % C

    q_sum = lax.dot_general(onehot, q, (((1,), (0,)), ((), ())),
                            precision=_HIGH,
                            preferred_element_type=jnp.float32)
    qg = q_sum / counts
    temp = jnp.float32(1.0 / sqrt(E))
    logits = lax.dot_general(qg, k, (((1,), (1,)), ((), ())),
                             preferred_element_type=jnp.float32) * temp
    lmax = jnp.max(logits, axis=1, keepdims=True)
    p = jnp.exp(logits - lmax)
    a = p / jnp.sum(p, axis=1, keepdims=True)
    vc = jnp.dot(a, v, preferred_element_type=jnp.float32)

    vc_ref[0] = vc
    assign_ref[0] = assign


def _tc_cluster_attend(q, k, v, planes_aug, bias_aug, sel):
    NH, L, E = q.shape
    return pl.pallas_call(
        _tc_body,
        grid=(NH,),
        in_specs=[
            pl.BlockSpec((1, L, E), lambda i: (i, 0, 0)),
            pl.BlockSpec((1, L, E), lambda i: (i, 0, 0)),
            pl.BlockSpec((1, L, E), lambda i: (i, 0, 0)),
            pl.BlockSpec((E, _BP), lambda i: (0, 0)),
            pl.BlockSpec((8, _BP), lambda i: (0, 0)),
            pl.BlockSpec((_CLUSTERS, L), lambda i: (0, 0)),
        ],
        out_specs=[
            pl.BlockSpec((1, _CLUSTERS, E), lambda i: (i, 0, 0)),
            pl.BlockSpec((1, 1, L), lambda i: (i, 0, 0)),
        ],
        out_shape=[
            jax.ShapeDtypeStruct((NH, _CLUSTERS, E), jnp.float32),
            jax.ShapeDtypeStruct((NH, 1, L), jnp.int32),
        ],
    )(q, k, v, planes_aug, bias_aug, sel)


def _make_sc_gather(B, D):
    info = plsc.get_sparse_core_info()
    NC, NS = info.num_cores, info.num_subcores
    NW = NC * NS
    RPW = B // NW
    CH = 128
    NBUF = 4
    nch = RPW // CH
    mesh = plsc.VectorSubcoreMesh(core_axis_name="c", subcore_axis_name="s")

    @functools.partial(
        pl.kernel,
        mesh=mesh,
        compiler_params=pltpu.CompilerParams(use_tc_tiling_on_sc=False),
        out_type=jax.ShapeDtypeStruct((B, D), jnp.float32),
        scratch_types=[
            pltpu.VMEM((RPW,), jnp.int32),
            pltpu.VMEM((NBUF, CH, D), jnp.float32),
        ] + [pltpu.SemaphoreType.DMA] * (2 * NBUF),
    )
    def _sc_gather(table_hbm, idx_hbm, out_hbm, idx_v, rows_v, *sems):
        gsems, wsems = sems[:NBUF], sems[NBUF:]
        wid = lax.axis_index("s") * NC + lax.axis_index("c")
        base = wid * RPW
        pltpu.sync_copy(idx_hbm.at[pl.ds(base, RPW)], idx_v)

        def _start(c):
            return pltpu.async_copy(
                table_hbm.at[idx_v.at[pl.ds(c * CH, CH)]],
                rows_v.at[c % NBUF], gsems[c % NBUF])

        handles = {}
        wh = {}
        for b in range(min(NBUF, nch)):
            handles[b] = _start(b)
        for c in range(nch):
            handles.pop(c).wait()
            wh[c] = pltpu.async_copy(
                rows_v.at[c % NBUF],
                out_hbm.at[pl.ds(base + c * CH, CH)], wsems[c % NBUF])
            nxt = c + NBUF
            if nxt < nch:
                wh.pop(nxt - NBUF).wait()
                handles[nxt] = _start(nxt)
        for c in sorted(wh):
            wh.pop(c).wait()

    return _sc_gather


def kernel(queries, keys, values):
    N, L, H, E = queries.shape
    D = values.shape[-1]
    NH = N * H
    B = N * L * H

    q = jnp.transpose(queries, (0, 2, 1, 3)).reshape(NH, L, E)
    k = jnp.transpose(keys, (0, 2, 1, 3)).reshape(NH, L, E)
    v = jnp.transpose(values, (0, 2, 1, 3)).reshape(NH, L, D)

    planes = jax.random.normal(jax.random.key(42), (_BITS, E + 1), dtype=jnp.float32)
    pad = _BP - _BITS
    planes_aug = jnp.concatenate(
        [planes[:, :-1].T, jnp.zeros((E, pad), jnp.float32)], axis=1)
    bias_aug = jnp.tile(
        jnp.concatenate([planes[:, -1], jnp.ones((pad,), jnp.float32)])[None, :],
        (8, 1))
    init_idx = jnp.linspace(0, L - 1, _CLUSTERS).astype(jnp.int32)
    sel = (init_idx[:, None] == jnp.arange(L, dtype=jnp.int32)[None, :]
           ).astype(jnp.float32)

    vc, assign = _tc_cluster_attend(q, k, v, planes_aug, bias_aug, sel)

    head_off = (jnp.arange(NH, dtype=jnp.int32) * _CLUSTERS).reshape(N, H, 1)
    idx = jnp.transpose(assign.reshape(N, H, L) + head_off, (0, 2, 1)).reshape(B)

    out = _make_sc_gather(B, D)(vc.reshape(NH * _CLUSTERS, D), idx)
    return out.reshape(N, L, H, D)
